# baseline (speedup 1.0000x reference)
import functools
import math

import jax
import jax.numpy as jnp
from jax import lax
from jax.experimental import pallas as pl
from jax.experimental.pallas import tpu as pltpu
from jax.experimental.pallas import tpu_sc as plsc

D_MODEL = 512
SCALE = math.sqrt(D_MODEL)

NC = 2
NS = 16
NW = NC * NS
LANES = 16

CHUNK = 128


def _make_gather_kernel(B: int):
    assert B % (8 * NW) == 0
    b_per_w = B // NW
    assert b_per_w % CHUNK == 0
    n_chunks = b_per_w // CHUNK

    mesh = plsc.VectorSubcoreMesh(core_axis_name="c", subcore_axis_name="s")

    @functools.partial(
        pl.kernel,
        mesh=mesh,
        out_type=jax.ShapeDtypeStruct((B, D_MODEL), jnp.float32),
        scratch_types=[
            pltpu.VMEM((CHUNK,), jnp.int32),
            pltpu.VMEM((CHUNK, D_MODEL), jnp.float32),
            pltpu.SemaphoreType.DMA,
        ],
    )
    def gather_scale(table_hbm, idx_hbm, out_hbm, idx_v, rows_v, sem):
        wid = lax.axis_index("s") * NC + lax.axis_index("c")
        w_base = wid * b_per_w

        def chunk_body(g, carry):
            base = pl.multiple_of(w_base + g * CHUNK, 8)
            pltpu.sync_copy(idx_hbm.at[pl.ds(base, CHUNK)], idx_v)
            pltpu.async_copy(table_hbm.at[idx_v], rows_v, sem).wait()

            def row_body(r, c2):
                for i in range(D_MODEL // LANES):
                    sl = pl.ds(i * LANES, LANES)
                    rows_v[r, sl] = rows_v[r, sl] * SCALE
                return c2

            lax.fori_loop(0, CHUNK, row_body, 0)
            pltpu.sync_copy(rows_v, out_hbm.at[pl.ds(base, CHUNK)])
            return carry

        lax.fori_loop(0, n_chunks, chunk_body, 0)

    return gather_scale


def kernel(x, table):
    B = x.size
    flat_idx = x.reshape((B,)).astype(jnp.int32)
    out = _make_gather_kernel(B)(table, flat_idx)
    return out.reshape(x.shape + (D_MODEL,))


# speedup vs baseline: 1.1375x; 1.1375x over previous
import functools
import math

import jax
import jax.numpy as jnp
from jax import lax
from jax.experimental import pallas as pl
from jax.experimental.pallas import tpu as pltpu
from jax.experimental.pallas import tpu_sc as plsc

D_MODEL = 512
SCALE = math.sqrt(D_MODEL)

NC = 2
NS = 16
NW = NC * NS
LANES = 16

CHUNK = 64


def _make_gather_kernel(B: int):
    assert B % (8 * NW) == 0
    b_per_w = B // NW
    assert b_per_w % (2 * CHUNK) == 0
    n_pairs = b_per_w // (2 * CHUNK)

    mesh = plsc.VectorSubcoreMesh(core_axis_name="c", subcore_axis_name="s")

    @functools.partial(
        pl.kernel,
        mesh=mesh,
        out_type=jax.ShapeDtypeStruct((B, D_MODEL), jnp.float32),
        scratch_types=[
            pltpu.VMEM((CHUNK,), jnp.int32),
            pltpu.VMEM((CHUNK,), jnp.int32),
            pltpu.VMEM((CHUNK, D_MODEL), jnp.float32),
            pltpu.VMEM((CHUNK, D_MODEL), jnp.float32),
            pltpu.SemaphoreType.DMA,
            pltpu.SemaphoreType.DMA,
            pltpu.SemaphoreType.DMA,
            pltpu.SemaphoreType.DMA,
        ],
    )
    def gather_scale(table_hbm, idx_hbm, out_hbm,
                     idx0, idx1, rows0, rows1, sg0, sg1, ss0, ss1):
        wid = lax.axis_index("s") * NC + lax.axis_index("c")
        w_base = wid * b_per_w
        idx = (idx0, idx1)
        rows = (rows0, rows1)
        sg = (sg0, sg1)
        ss = (ss0, ss1)

        def start_gather(b, g):
            base = pl.multiple_of(w_base + g * CHUNK, 8)
            pltpu.sync_copy(idx_hbm.at[pl.ds(base, CHUNK)], idx[b])
            pltpu.async_copy(table_hbm.at[idx[b]], rows[b], sg[b])

        def wait_gather(b):
            pltpu.make_async_copy(table_hbm.at[idx[b]], rows[b], sg[b]).wait()

        def scale_rows(b):
            def row_body(r, c2):
                for i in range(D_MODEL // LANES):
                    sl = pl.ds(i * LANES, LANES)
                    rows[b][r, sl] = rows[b][r, sl] * SCALE
                return c2

            lax.fori_loop(0, CHUNK, row_body, 0)

        def start_scatter(b, g):
            base = pl.multiple_of(w_base + g * CHUNK, 8)
            pltpu.async_copy(rows[b], out_hbm.at[pl.ds(base, CHUNK)], ss[b])

        def wait_scatter(b, g):
            base = pl.multiple_of(w_base + g * CHUNK, 8)
            pltpu.make_async_copy(rows[b], out_hbm.at[pl.ds(base, CHUNK)], ss[b]).wait()

        start_gather(0, 0)
        start_gather(1, 1)

        def pair_body(k, carry):
            g0 = 2 * k
            for b in (0, 1):
                wait_gather(b)
                scale_rows(b)
                start_scatter(b, g0 + b)
            for b in (0, 1):
                @pl.when(k < n_pairs - 1)
                def _(b=b):
                    wait_scatter(b, g0 + b)
                    start_gather(b, g0 + b + 2)
            return carry

        lax.fori_loop(0, n_pairs, pair_body, 0)
        wait_scatter(0, 2 * n_pairs - 2)
        wait_scatter(1, 2 * n_pairs - 1)

    return gather_scale


def kernel(x, table):
    B = x.size
    flat_idx = x.reshape((B,)).astype(jnp.int32)
    out = _make_gather_kernel(B)(table, flat_idx)
    return out.reshape(x.shape + (D_MODEL,))


# speedup vs baseline: 3.4935x; 3.0712x over previous
import functools
import math

import jax
import jax.numpy as jnp
from jax import lax
from jax.experimental import pallas as pl
from jax.experimental.pallas import tpu as pltpu
from jax.experimental.pallas import tpu_sc as plsc

D_MODEL = 512
SCALE = math.sqrt(D_MODEL)

NC = 2
NS = 16
NW = NC * NS
LANES = 16

CHUNK = 64


def _make_gather_kernel(B: int):
    assert B % (8 * NW) == 0
    b_per_w = B // NW
    assert b_per_w % (2 * CHUNK) == 0
    n_pairs = b_per_w // (2 * CHUNK)

    mesh = plsc.VectorSubcoreMesh(core_axis_name="c", subcore_axis_name="s")

    @functools.partial(
        pl.kernel,
        mesh=mesh,
        out_type=jax.ShapeDtypeStruct((B, D_MODEL), jnp.float32),
        scratch_types=[
            pltpu.VMEM((CHUNK,), jnp.int32),
            pltpu.VMEM((CHUNK,), jnp.int32),
            pltpu.VMEM((CHUNK, D_MODEL), jnp.float32),
            pltpu.VMEM((CHUNK, D_MODEL), jnp.float32),
            pltpu.SemaphoreType.DMA,
            pltpu.SemaphoreType.DMA,
            pltpu.SemaphoreType.DMA,
            pltpu.SemaphoreType.DMA,
        ],
    )
    def gather_scale(table_hbm, idx_hbm, out_hbm,
                     idx0, idx1, rows0, rows1, sg0, sg1, ss0, ss1):
        wid = lax.axis_index("s") * NC + lax.axis_index("c")
        w_base = wid * b_per_w
        idx = (idx0, idx1)
        rows = (rows0, rows1)
        sg = (sg0, sg1)
        ss = (ss0, ss1)

        def start_gather(b, g):
            base = pl.multiple_of(w_base + g * CHUNK, 8)
            pltpu.sync_copy(idx_hbm.at[pl.ds(base, CHUNK)], idx[b])
            pltpu.async_copy(table_hbm.at[idx[b]], rows[b], sg[b])

        def wait_gather(b):
            pltpu.make_async_copy(table_hbm.at[idx[b]], rows[b], sg[b]).wait()

        def scale_rows(b):
            def row_body(r, c2):
                for i in range(D_MODEL // LANES):
                    sl = pl.ds(i * LANES, LANES)
                    rows[b][r, sl] = rows[b][r, sl] * SCALE
                return c2

            lax.fori_loop(0, CHUNK, row_body, 0)

        def start_scatter(b, g):
            base = pl.multiple_of(w_base + g * CHUNK, 8)
            pltpu.async_copy(rows[b], out_hbm.at[pl.ds(base, CHUNK)], ss[b])

        def wait_scatter(b, g):
            base = pl.multiple_of(w_base + g * CHUNK, 8)
            pltpu.make_async_copy(rows[b], out_hbm.at[pl.ds(base, CHUNK)], ss[b]).wait()

        start_gather(0, 0)
        start_gather(1, 1)

        def pair_body(k, carry):
            g0 = 2 * k
            for b in (0, 1):
                wait_gather(b)
                scale_rows(b)
                start_scatter(b, g0 + b)
            for b in (0, 1):
                @pl.when(k < n_pairs - 1)
                def _(b=b):
                    wait_scatter(b, g0 + b)
                    start_gather(b, g0 + b + 2)
            return carry

        lax.fori_loop(0, n_pairs, pair_body, 0)
        wait_scatter(0, 2 * n_pairs - 2)
        wait_scatter(1, 2 * n_pairs - 1)

    return gather_scale


def kernel(x, table):
    B = x.size
    flat_idx = x.T.reshape((B,)).astype(jnp.int32)
    out = _make_gather_kernel(B)(table, flat_idx)
    out3 = out.reshape((x.shape[1], x.shape[0], D_MODEL))
    return out3.transpose(1, 0, 2)


# speedup vs baseline: 3.5380x; 1.0127x over previous
import functools
import math

import jax
import jax.numpy as jnp
from jax import lax
from jax.experimental import pallas as pl
from jax.experimental.pallas import tpu as pltpu
from jax.experimental.pallas import tpu_sc as plsc

D_MODEL = 512
SCALE = math.sqrt(D_MODEL)

NC = 2
NS = 16
NW = NC * NS
LANES = 16

CHUNK = 80


def _make_gather_kernel(B: int):
    assert B % (8 * NW) == 0
    b_per_w = B // NW
    assert b_per_w % (2 * CHUNK) == 0
    n_pairs = b_per_w // (2 * CHUNK)

    mesh = plsc.VectorSubcoreMesh(core_axis_name="c", subcore_axis_name="s")

    @functools.partial(
        pl.kernel,
        mesh=mesh,
        out_type=jax.ShapeDtypeStruct((B, D_MODEL), jnp.float32),
        scratch_types=[
            pltpu.VMEM((b_per_w,), jnp.int32),
            pltpu.VMEM((CHUNK, D_MODEL), jnp.float32),
            pltpu.VMEM((CHUNK, D_MODEL), jnp.float32),
            pltpu.SemaphoreType.DMA,
            pltpu.SemaphoreType.DMA,
            pltpu.SemaphoreType.DMA,
            pltpu.SemaphoreType.DMA,
        ],
    )
    def gather_scale(table_hbm, idx_hbm, out_hbm,
                     idx_all, rows0, rows1, sg0, sg1, ss0, ss1):
        wid = lax.axis_index("s") * NC + lax.axis_index("c")
        w_base = wid * b_per_w
        rows = (rows0, rows1)
        sg = (sg0, sg1)
        ss = (ss0, ss1)

        pltpu.sync_copy(idx_hbm.at[pl.ds(pl.multiple_of(w_base, 8), b_per_w)],
                        idx_all)

        def start_gather(b, g):
            off = pl.multiple_of(g * CHUNK, 8)
            pltpu.async_copy(table_hbm.at[idx_all.at[pl.ds(off, CHUNK)]],
                             rows[b], sg[b])

        def wait_gather(b, g):
            off = pl.multiple_of(g * CHUNK, 8)
            pltpu.make_async_copy(table_hbm.at[idx_all.at[pl.ds(off, CHUNK)]],
                                  rows[b], sg[b]).wait()

        def scale_rows(b):
            def row_body(r, c2):
                for i in range(D_MODEL // LANES):
                    sl = pl.ds(i * LANES, LANES)
                    rows[b][r, sl] = rows[b][r, sl] * SCALE
                return c2

            lax.fori_loop(0, CHUNK, row_body, 0)

        def start_scatter(b, g):
            base = pl.multiple_of(w_base + g * CHUNK, 8)
            pltpu.async_copy(rows[b], out_hbm.at[pl.ds(base, CHUNK)], ss[b])

        def wait_scatter(b, g):
            base = pl.multiple_of(w_base + g * CHUNK, 8)
            pltpu.make_async_copy(rows[b], out_hbm.at[pl.ds(base, CHUNK)], ss[b]).wait()

        start_gather(0, 0)
        start_gather(1, 1)

        def pair_body(k, carry):
            g0 = 2 * k
            for b in (0, 1):
                wait_gather(b, g0 + b)
                scale_rows(b)
                start_scatter(b, g0 + b)
            for b in (0, 1):
                @pl.when(k < n_pairs - 1)
                def _(b=b):
                    wait_scatter(b, g0 + b)
                    start_gather(b, g0 + b + 2)
            return carry

        lax.fori_loop(0, n_pairs, pair_body, 0)
        wait_scatter(0, 2 * n_pairs - 2)
        wait_scatter(1, 2 * n_pairs - 1)

    return gather_scale


def kernel(x, table):
    B = x.size
    flat_idx = x.T.reshape((B,)).astype(jnp.int32)
    out = _make_gather_kernel(B)(table, flat_idx)
    out3 = out.reshape((x.shape[1], x.shape[0], D_MODEL))
    return out3.transpose(1, 0, 2)


# speedup vs baseline: 3.5822x; 1.0125x over previous
import functools
import math

import jax
import jax.numpy as jnp
from jax import lax
from jax.experimental import pallas as pl
from jax.experimental.pallas import tpu as pltpu
from jax.experimental.pallas import tpu_sc as plsc

D_MODEL = 512
SCALE = math.sqrt(D_MODEL)

NC = 2
NS = 16
NW = NC * NS
LANES = 16

CHUNK = 40


def _make_gather_kernel(B: int):
    assert B % (8 * NW) == 0
    b_per_w = B // NW
    assert b_per_w % (2 * CHUNK) == 0
    n_pairs = b_per_w // (2 * CHUNK)

    mesh = plsc.VectorSubcoreMesh(core_axis_name="c", subcore_axis_name="s")

    @functools.partial(
        pl.kernel,
        mesh=mesh,
        out_type=jax.ShapeDtypeStruct((B, D_MODEL), jnp.float32),
        scratch_types=[
            pltpu.VMEM((b_per_w,), jnp.int32),
            pltpu.VMEM((CHUNK, D_MODEL), jnp.float32),
            pltpu.VMEM((CHUNK, D_MODEL), jnp.float32),
            pltpu.VMEM((CHUNK, D_MODEL), jnp.float32),
            pltpu.VMEM((CHUNK, D_MODEL), jnp.float32),
            pltpu.SemaphoreType.DMA,
            pltpu.SemaphoreType.DMA,
            pltpu.SemaphoreType.DMA,
            pltpu.SemaphoreType.DMA,
        ],
    )
    def gather_scale(table_hbm, idx_hbm, out_hbm,
                     idx_all, in0, in1, out0, out1, sg0, sg1, ss0, ss1):
        wid = lax.axis_index("s") * NC + lax.axis_index("c")
        w_base = wid * b_per_w
        ibuf = (in0, in1)
        obuf = (out0, out1)
        sg = (sg0, sg1)
        ss = (ss0, ss1)

        pltpu.sync_copy(idx_hbm.at[pl.ds(pl.multiple_of(w_base, 8), b_per_w)],
                        idx_all)

        def start_gather(b, g):
            off = pl.multiple_of(g * CHUNK, 8)
            pltpu.async_copy(table_hbm.at[idx_all.at[pl.ds(off, CHUNK)]],
                             ibuf[b], sg[b])

        def wait_gather(b, g):
            off = pl.multiple_of(g * CHUNK, 8)
            pltpu.make_async_copy(table_hbm.at[idx_all.at[pl.ds(off, CHUNK)]],
                                  ibuf[b], sg[b]).wait()

        def scale_rows(b):
            def row_body(r, c2):
                for i in range(D_MODEL // LANES):
                    sl = pl.ds(i * LANES, LANES)
                    obuf[b][r, sl] = ibuf[b][r, sl] * SCALE
                return c2

            lax.fori_loop(0, CHUNK, row_body, 0)

        def start_scatter(b, g):
            base = pl.multiple_of(w_base + g * CHUNK, 8)
            pltpu.async_copy(obuf[b], out_hbm.at[pl.ds(base, CHUNK)], ss[b])

        def wait_scatter(b, g):
            base = pl.multiple_of(w_base + g * CHUNK, 8)
            pltpu.make_async_copy(obuf[b], out_hbm.at[pl.ds(base, CHUNK)], ss[b]).wait()

        start_gather(0, 0)
        start_gather(1, 1)

        def pair_body(k, carry):
            g0 = 2 * k
            for b in (0, 1):
                g = g0 + b
                wait_gather(b, g)

                @pl.when(k > 0)
                def _(b=b, g=g):
                    wait_scatter(b, g - 2)

                scale_rows(b)
                start_scatter(b, g)

                @pl.when(k < n_pairs - 1)
                def _(b=b, g=g):
                    start_gather(b, g + 2)
            return carry

        lax.fori_loop(0, n_pairs, pair_body, 0)
        wait_scatter(0, 2 * n_pairs - 2)
        wait_scatter(1, 2 * n_pairs - 1)

    return gather_scale


def kernel(x, table):
    B = x.size
    flat_idx = x.T.reshape((B,)).astype(jnp.int32)
    out = _make_gather_kernel(B)(table, flat_idx)
    out3 = out.reshape((x.shape[1], x.shape[0], D_MODEL))
    return out3.transpose(1, 0, 2)


# speedup vs baseline: 3.5915x; 1.0026x over previous
import functools
import math

import jax
import jax.numpy as jnp
from jax import lax
from jax.experimental import pallas as pl
from jax.experimental.pallas import tpu as pltpu
from jax.experimental.pallas import tpu_sc as plsc

D_MODEL = 512
SCALE = math.sqrt(D_MODEL)

NC = 2
NS = 16
NW = NC * NS
LANES = 16

CHUNK = 16
NBUF = 4


def _make_gather_kernel(B: int):
    assert B % (8 * NW) == 0
    b_per_w = B // NW
    assert b_per_w % (NBUF * CHUNK) == 0
    n_iters = b_per_w // (NBUF * CHUNK)

    mesh = plsc.VectorSubcoreMesh(core_axis_name="c", subcore_axis_name="s")

    @functools.partial(
        pl.kernel,
        mesh=mesh,
        out_type=jax.ShapeDtypeStruct((B, D_MODEL), jnp.float32),
        scratch_types=(
            [pltpu.VMEM((b_per_w,), jnp.int32)]
            + [pltpu.VMEM((CHUNK, D_MODEL), jnp.float32)] * (2 * NBUF)
            + [pltpu.SemaphoreType.DMA] * (2 * NBUF)
        ),
    )
    def gather_scale(table_hbm, idx_hbm, out_hbm, idx_all, *bufs_and_sems):
        ibuf = bufs_and_sems[0:NBUF]
        obuf = bufs_and_sems[NBUF:2 * NBUF]
        sg = bufs_and_sems[2 * NBUF:3 * NBUF]
        ss = bufs_and_sems[3 * NBUF:4 * NBUF]
        wid = lax.axis_index("s") * NC + lax.axis_index("c")
        w_base = wid * b_per_w

        pltpu.sync_copy(idx_hbm.at[pl.ds(pl.multiple_of(w_base, 8), b_per_w)],
                        idx_all)

        def start_gather(b, g):
            off = pl.multiple_of(g * CHUNK, 8)
            pltpu.async_copy(table_hbm.at[idx_all.at[pl.ds(off, CHUNK)]],
                             ibuf[b], sg[b])

        def wait_gather(b, g):
            off = pl.multiple_of(g * CHUNK, 8)
            pltpu.make_async_copy(table_hbm.at[idx_all.at[pl.ds(off, CHUNK)]],
                                  ibuf[b], sg[b]).wait()

        def scale_rows(b):
            def row_body(r, c2):
                for i in range(D_MODEL // LANES):
                    sl = pl.ds(i * LANES, LANES)
                    obuf[b][r, sl] = ibuf[b][r, sl] * SCALE
                return c2

            lax.fori_loop(0, CHUNK, row_body, 0)

        def start_scatter(b, g):
            base = pl.multiple_of(w_base + g * CHUNK, 8)
            pltpu.async_copy(obuf[b], out_hbm.at[pl.ds(base, CHUNK)], ss[b])

        def wait_scatter(b, g):
            base = pl.multiple_of(w_base + g * CHUNK, 8)
            pltpu.make_async_copy(obuf[b], out_hbm.at[pl.ds(base, CHUNK)], ss[b]).wait()

        for b in range(NBUF):
            start_gather(b, b)

        def iter_body(k, carry):
            g0 = NBUF * k
            for b in range(NBUF):
                g = g0 + b
                wait_gather(b, g)

                @pl.when(k > 0)
                def _(b=b, g=g):
                    wait_scatter(b, g - NBUF)

                scale_rows(b)
                start_scatter(b, g)

                @pl.when(k < n_iters - 1)
                def _(b=b, g=g):
                    start_gather(b, g + NBUF)
            return carry

        lax.fori_loop(0, n_iters, iter_body, 0)
        for b in range(NBUF):
            wait_scatter(b, NBUF * (n_iters - 1) + b)

    return gather_scale


def kernel(x, table):
    B = x.size
    flat_idx = x.T.reshape((B,)).astype(jnp.int32)
    out = _make_gather_kernel(B)(table, flat_idx)
    out3 = out.reshape((x.shape[1], x.shape[0], D_MODEL))
    return out3.transpose(1, 0, 2)


# speedup vs baseline: 3.6010x; 1.0027x over previous
import functools
import math

import jax
import jax.numpy as jnp
from jax import lax
from jax.experimental import pallas as pl
from jax.experimental.pallas import tpu as pltpu
from jax.experimental.pallas import tpu_sc as plsc

D_MODEL = 512
SCALE = math.sqrt(D_MODEL)

NC = 2
NS = 16
NW = NC * NS
LANES = 16

CHUNK = 32
NBUF = 2


def _make_gather_kernel(B: int):
    assert B % (8 * NW) == 0
    b_per_w = B // NW
    assert b_per_w % (NBUF * CHUNK) == 0
    n_iters = b_per_w // (NBUF * CHUNK)

    mesh = plsc.VectorSubcoreMesh(core_axis_name="c", subcore_axis_name="s")

    @functools.partial(
        pl.kernel,
        mesh=mesh,
        out_type=jax.ShapeDtypeStruct((B, D_MODEL), jnp.float32),
        scratch_types=(
            [pltpu.VMEM((b_per_w,), jnp.int32)]
            + [pltpu.VMEM((CHUNK, D_MODEL), jnp.float32)] * (2 * NBUF)
            + [pltpu.SemaphoreType.DMA] * (2 * NBUF)
        ),
    )
    def gather_scale(table_hbm, idx_hbm, out_hbm, idx_all, *bufs_and_sems):
        ibuf = bufs_and_sems[0:NBUF]
        obuf = bufs_and_sems[NBUF:2 * NBUF]
        sg = bufs_and_sems[2 * NBUF:3 * NBUF]
        ss = bufs_and_sems[3 * NBUF:4 * NBUF]
        wid = lax.axis_index("s") * NC + lax.axis_index("c")
        w_base = wid * b_per_w

        pltpu.sync_copy(idx_hbm.at[pl.ds(pl.multiple_of(w_base, 8), b_per_w)],
                        idx_all)

        def start_gather(b, g):
            off = pl.multiple_of(g * CHUNK, 8)
            pltpu.async_copy(table_hbm.at[idx_all.at[pl.ds(off, CHUNK)]],
                             ibuf[b], sg[b])

        def wait_gather(b, g):
            off = pl.multiple_of(g * CHUNK, 8)
            pltpu.make_async_copy(table_hbm.at[idx_all.at[pl.ds(off, CHUNK)]],
                                  ibuf[b], sg[b]).wait()

        def scale_rows(b):
            def row_body(r, c2):
                for i in range(D_MODEL // LANES):
                    sl = pl.ds(i * LANES, LANES)
                    obuf[b][r, sl] = ibuf[b][r, sl] * SCALE
                return c2

            lax.fori_loop(0, CHUNK, row_body, 0)

        def start_scatter(b, g):
            base = pl.multiple_of(w_base + g * CHUNK, 8)
            pltpu.async_copy(obuf[b], out_hbm.at[pl.ds(base, CHUNK)], ss[b])

        def wait_scatter(b, g):
            base = pl.multiple_of(w_base + g * CHUNK, 8)
            pltpu.make_async_copy(obuf[b], out_hbm.at[pl.ds(base, CHUNK)], ss[b]).wait()

        for b in range(NBUF):
            start_gather(b, b)

        def iter_body(k, carry):
            g0 = NBUF * k
            for b in range(NBUF):
                g = g0 + b
                wait_gather(b, g)

                @pl.when(k > 0)
                def _(b=b, g=g):
                    wait_scatter(b, g - NBUF)

                scale_rows(b)
                start_scatter(b, g)

                @pl.when(k < n_iters - 1)
                def _(b=b, g=g):
                    start_gather(b, g + NBUF)
            return carry

        lax.fori_loop(0, n_iters, iter_body, 0)
        for b in range(NBUF):
            wait_scatter(b, NBUF * (n_iters - 1) + b)

    return gather_scale


def kernel(x, table):
    B = x.size
    flat_idx = x.T.reshape((B,)).astype(jnp.int32)
    out = _make_gather_kernel(B)(table, flat_idx)
    out3 = out.reshape((x.shape[1], x.shape[0], D_MODEL))
    return out3.transpose(1, 0, 2)
